# Initial kernel scaffold; baseline (speedup 1.0000x reference)
import jax
import jax.numpy as jnp
from jax.experimental import pallas as pl


def kernel(z_mask, pos_l3, pos_l2, pos_l1, pos_l0, batch_l3, batch_l2, batch_l1, batch_l0, src1, dst1, src2, dst2, src3, dst3, W1a, b1a, W1b, b1b, W2a, b2a, W2b, b2b, W3a, b3a, W3b, b3b, Wlin, blin):
    raise NotImplementedError("write your pallas kernel here")


# speedup vs baseline: 8.0743x; 8.0743x over previous
import functools

import jax
import jax.numpy as jnp
from jax import lax
from jax.experimental import pallas as pl
from jax.experimental.pallas import tpu as pltpu
from jax.experimental.pallas import tpu_sc as plsc

_NC = 2
_NS = 16
_NW = _NC * _NS

_PADE = 2016
_CBK = 2000


def _celu(x):
    return jnp.where(x > 0, x, jnp.exp(x) - 1.0)


def _node_precompute(x, pos_in, wtop, wpos, ba, n_rows, apply_celu):
    cin, cmid = wtop.shape
    blk = 1024

    def body(x_ref, p_ref, wt_ref, wp_ref, b_ref, o_ref):
        xv = x_ref[...]
        if apply_celu:
            xv = _celu(xv)
        acc = jnp.dot(xv, wt_ref[...], preferred_element_type=jnp.float32)
        acc = acc + jnp.dot(p_ref[...], wp_ref[...],
                            preferred_element_type=jnp.float32)
        o_ref[...] = acc + b_ref[...]

    return pl.pallas_call(
        body,
        grid=(pl.cdiv(n_rows, blk),),
        in_specs=[
            pl.BlockSpec((blk, cin), lambda i: (i, 0)),
            pl.BlockSpec((blk, 2), lambda i: (i, 0)),
            pl.BlockSpec((cin, cmid), lambda i: (0, 0)),
            pl.BlockSpec((2, cmid), lambda i: (0, 0)),
            pl.BlockSpec((1, cmid), lambda i: (0, 0)),
        ],
        out_specs=pl.BlockSpec((blk, cmid), lambda i: (i, 0)),
        out_shape=jax.ShapeDtypeStruct((n_rows, cmid), jnp.float32),
    )(x, pos_in, wtop, wpos, ba.reshape(1, cmid))


def _edge_mlp(qa2, posd2, wpos2, wb2, bias2):
    rows, _ = qa2.shape
    pk = posd2.shape[1]
    wk = wb2.shape[1]
    blk = 1024

    def body(qa_ref, pd_ref, wp_ref, wb_ref, b_ref, o_ref):
        g = qa_ref[...] - jnp.dot(pd_ref[...], wp_ref[...],
                                  preferred_element_type=jnp.float32)
        g = jnp.maximum(g, 0.0)
        o_ref[...] = jnp.dot(g, wb_ref[...],
                             preferred_element_type=jnp.float32) + b_ref[...]

    return pl.pallas_call(
        body,
        grid=(pl.cdiv(rows, blk),),
        in_specs=[
            pl.BlockSpec((blk, 128), lambda i: (i, 0)),
            pl.BlockSpec((blk, pk), lambda i: (i, 0)),
            pl.BlockSpec((pk, 128), lambda i: (0, 0)),
            pl.BlockSpec((128, wk), lambda i: (0, 0)),
            pl.BlockSpec((1, wk), lambda i: (0, 0)),
        ],
        out_specs=pl.BlockSpec((blk, wk), lambda i: (i, 0)),
        out_shape=jax.ShapeDtypeStruct((rows, wk), jnp.float32),
    )(qa2, posd2, wpos2, wb2, bias2.reshape(1, wk))


def _head(x, wlin, blin):
    rows, cin = x.shape
    blk = 2048

    def body(x_ref, w_ref, b_ref, o_ref):
        z = jnp.dot(_celu(x_ref[...]), w_ref[...],
                    preferred_element_type=jnp.float32) + b_ref[...]
        o_ref[...] = jnp.minimum(z, 0.0) - jnp.log(1.0 + jnp.exp(-jnp.abs(z)))

    return pl.pallas_call(
        body,
        grid=(pl.cdiv(rows, blk),),
        in_specs=[
            pl.BlockSpec((blk, cin), lambda i: (i, 0)),
            pl.BlockSpec((cin, 1), lambda i: (0, 0)),
            pl.BlockSpec((1, 1), lambda i: (0, 0)),
        ],
        out_specs=pl.BlockSpec((blk, 1), lambda i: (i, 0)),
        out_shape=jax.ShapeDtypeStruct((rows, 1), jnp.float32),
    )(x, wlin, blin.reshape(1, 1))


def _sc_gather(a_nodes, pos_out, src, dst, n_edges, k):
    cmid = a_nodes.shape[1]
    ew = n_edges // _NW
    cb = 1024
    nbig = ew // cb
    tail = ew - nbig * cb
    mesh = plsc.VectorSubcoreMesh(core_axis_name="c", subcore_axis_name="s", num_cores=_NC, num_subcores=_NS)

    @functools.partial(
        pl.kernel,
        out_type=[
            jax.ShapeDtypeStruct((n_edges + _PADE, cmid), jnp.float32),
            jax.ShapeDtypeStruct((n_edges + _PADE, 2), jnp.float32),
        ],
        mesh=mesh,
        compiler_params=pltpu.CompilerParams(use_tc_tiling_on_sc=False),
        scratch_types=[
            pltpu.VMEM((cb,), jnp.int32),
            pltpu.VMEM((cb,), jnp.int32),
            pltpu.VMEM((cb, cmid), jnp.float32),
            pltpu.VMEM((cb, 2), jnp.float32),
            pltpu.SemaphoreType.DMA,
            pltpu.SemaphoreType.DMA,
        ],
    )
    def k2(a_hbm, po_hbm, src_hbm, dst_hbm, qa_hbm, pd_hbm,
           sidx, didx, arow, prow, sem1, sem2):
        wid = lax.axis_index("s") * _NC + lax.axis_index("c")
        base = wid * ew

        def do_chunk(off, n):
            pltpu.sync_copy(src_hbm.at[pl.ds(off, n)], sidx.at[pl.ds(0, n)])
            pltpu.sync_copy(dst_hbm.at[pl.ds(off, n)], didx.at[pl.ds(0, n)])
            copies = []
            j0 = 0
            while j0 < n:
                m = min(128, n - j0)
                copies.append(pltpu.async_copy(
                    a_hbm.at[sidx.at[pl.ds(j0, m)]],
                    arow.at[pl.ds(j0, m)], sem1))
                copies.append(pltpu.async_copy(
                    po_hbm.at[didx.at[pl.ds(j0, m)]],
                    prow.at[pl.ds(j0, m)], sem2))
                j0 += m
            for cpy in copies:
                cpy.wait()
            pltpu.sync_copy(arow.at[pl.ds(0, n)], qa_hbm.at[pl.ds(off, n)])
            pltpu.sync_copy(prow.at[pl.ds(0, n)], pd_hbm.at[pl.ds(off, n)])

        def big(c, carry):
            do_chunk(base + c * cb, cb)
            return carry

        lax.fori_loop(0, nbig, big, jnp.int32(0))
        if tail:
            do_chunk(base + nbig * cb, tail)

    return k2(a_nodes, pos_out, src, dst)


def _sc_segmax(p_edges, dst, bounds, n_edges, n_out, wn, width):
    nv = width // 16
    mesh = plsc.VectorSubcoreMesh(core_axis_name="c", subcore_axis_name="s", num_cores=_NC, num_subcores=_NS)

    @functools.partial(
        pl.kernel,
        out_type=jax.ShapeDtypeStruct((_NW * wn, width), jnp.float32),
        mesh=mesh,
        compiler_params=pltpu.CompilerParams(use_tc_tiling_on_sc=False),
        scratch_types=[
            pltpu.VMEM((_CBK, width), jnp.float32),
            pltpu.VMEM((_CBK,), jnp.int32),
            pltpu.VMEM((48,), jnp.int32),
            pltpu.VMEM((wn + 1, width), jnp.float32),
            pltpu.SemaphoreType.DMA,
        ],
    )
    def k3(p_hbm, dst_hbm, bnd_hbm, out_hbm, pchunk, dchunk, bnds, outbuf, sem):
        wid = lax.axis_index("s") * _NC + lax.axis_index("c")
        pltpu.sync_copy(bnd_hbm, bnds)
        bvec = bnds[pl.ds(wid, 16)]
        e_lo = bvec[0]
        e_hi = bvec[1]
        w_lo = wid * wn
        size = jnp.minimum(w_lo + wn, n_out) - w_lo

        zero = jnp.zeros((16,), jnp.float32)

        def zbody(i, carry):
            outbuf[i, pl.ds(0, 16)] = zero
            if nv == 2:
                outbuf[i, pl.ds(16, 16)] = zero
            return carry

        lax.fori_loop(0, wn, zbody, jnp.int32(0))

        e_start = (e_lo // 8) * 8
        nch = (e_hi - e_start + _CBK - 1) // _CBK

        def chunk_body(t, carry):
            off = e_start + t * _CBK
            pltpu.sync_copy(p_hbm.at[pl.ds(off, _CBK)], pchunk)
            pltpu.sync_copy(dst_hbm.at[pl.ds(off, _CBK)], dchunk)

            def grp(g, car):
                dvec = dchunk[pl.ds(g * 16, 16)]
                for j in range(16):
                    d_prev, a0, a1 = car
                    d = dvec[j]
                    i = g * 16 + j
                    r0 = pchunk[i, pl.ds(0, 16)]
                    r1 = pchunk[i, pl.ds(16, 16)] if nv == 2 else r0
                    is_new = d != d_prev
                    a0 = jnp.where(is_new, r0, jnp.maximum(a0, r0))
                    a1 = jnp.where(is_new, r1, jnp.maximum(a1, r1))
                    ru = d - w_lo
                    row = jnp.where((ru >= 0) & (ru < size), ru, wn)
                    outbuf[row, pl.ds(0, 16)] = a0
                    if nv == 2:
                        outbuf[row, pl.ds(16, 16)] = a1
                    car = (d, a0, a1)
                return car

            return lax.fori_loop(0, _CBK // 16, grp, carry)

        car0 = (jnp.int32(-1), zero, zero)
        lax.fori_loop(0, nch, chunk_body, car0)

        pltpu.sync_copy(outbuf.at[pl.ds(0, wn)], out_hbm.at[pl.ds(w_lo, wn)])

    return k3(p_edges, dst, bounds)


def _level(x, pos_in, pos_out, src, dst, wa, ba, wb, bb,
           n_in, n_out, apply_celu):
    n_edges = src.shape[0]
    cin = wa.shape[0] - 2
    cmid = wa.shape[1]
    cout = wb.shape[1]
    width = max(cout, 16)
    k = 128 // cmid
    wn = ((n_out + _NW - 1) // _NW + 7) // 8 * 8

    wtop = wa[:cin]
    wpos = wa[cin:]

    eye_k = jnp.eye(k, dtype=jnp.float32)
    wpos2 = jnp.kron(eye_k, wpos)
    wb_pad = jnp.pad(wb, ((0, 0), (0, width - cout)))
    wb2 = jnp.kron(eye_k, wb_pad)
    bias2 = jnp.tile(jnp.pad(bb, (0, width - cout)), k)

    wlos = jnp.minimum(jnp.arange(_NW + 1, dtype=jnp.int32) * wn, n_out)
    bounds = jnp.searchsorted(dst, wlos, side='left').astype(jnp.int32)
    bounds = jnp.concatenate([bounds, jnp.zeros((48 - _NW - 1,), jnp.int32)])

    dst_pad = jnp.concatenate(
        [dst, jnp.full((_PADE,), n_out, dtype=jnp.int32)])

    a_nodes = _node_precompute(x, pos_in, wtop, wpos, ba, n_in, apply_celu)
    qa, posd = _sc_gather(a_nodes, pos_out, src, dst_pad, n_edges, k)
    rows2 = (n_edges + _PADE) // k
    p2 = _edge_mlp(qa.reshape(rows2, 128), posd.reshape(rows2, 2 * k),
                   wpos2, wb2, bias2)
    p_edges = p2.reshape(n_edges + _PADE, width)
    return _sc_segmax(p_edges, dst_pad, bounds, n_edges, n_out, wn, width)


def kernel(z_mask, pos_l3, pos_l2, pos_l1, pos_l0,
           batch_l3, batch_l2, batch_l1, batch_l0,
           src1, dst1, src2, dst2, src3, dst3,
           W1a, b1a, W1b, b1b, W2a, b2a, W2b, b2b, W3a, b3a, W3b, b3b,
           Wlin, blin):
    del batch_l3, batch_l2, batch_l1, batch_l0
    n3, n2, n1, n0 = (pos_l3.shape[0], pos_l2.shape[0],
                      pos_l1.shape[0], pos_l0.shape[0])
    s1, d1 = src1.astype(jnp.int32), dst1.astype(jnp.int32)
    s2, d2 = src2.astype(jnp.int32), dst2.astype(jnp.int32)
    s3, d3 = src3.astype(jnp.int32), dst3.astype(jnp.int32)

    out1 = _level(z_mask, pos_l3, pos_l2, s1, d1, W1a, b1a, W1b, b1b,
                  n3, n2, apply_celu=False)
    out2 = _level(out1, pos_l2, pos_l1, s2, d2, W2a, b2a, W2b, b2b,
                  n2, n1, apply_celu=True)
    out3 = _level(out2, pos_l1, pos_l0, s3, d3, W3a, b3a, W3b, b3b,
                  n1, n0, apply_celu=True)

    wlin_pad = jnp.concatenate(
        [Wlin, jnp.zeros((16 - Wlin.shape[0], 1), jnp.float32)])
    y = _head(out3, wlin_pad, blin)
    return y[:n0]
